# baseline (speedup 1.0000x reference)
import functools
import jax
import jax.numpy as jnp
from jax import lax
from jax.experimental import pallas as pl
from jax.experimental.pallas import tpu as pltpu
from jax.experimental.pallas import tpu_sc as plsc

USER_N = 5000
ITEM_N = 5000
N_NODES = USER_N + ITEM_N
N_EDGES = 160000
EMB = 256
N_LAYERS = 3

CPT = 8
CPP = 4
NT = 32
SLAB = N_NODES * CPT
HSLAB = N_NODES * CPP
CHUNK = 1600
NCHUNK = N_EDGES // CHUNK
FR = 500
FE = FR * CPP


def _sc_body(ego0_hbm, src_hbm, dst_hbm, w_hbm,
             l1_hbm, l2_hbm, out_hbm,
             bufa, bufb, srcb, dstb, wb, fl1, fl2, fl3,
             esem):
    c = lax.axis_index("c")
    s = lax.axis_index("s")
    t = c * 16 + s
    toff = t * SLAB
    iota = lax.iota(jnp.int32, 16)
    three = jnp.full((16,), 3, jnp.int32)

    def rowcol(lf):
        return lax.shift_right_logical(lf, 2), lax.bitwise_and(lf, three)

    def fire_edges(g, p):
        off = g * CHUNK
        pltpu.make_async_copy(src_hbm.at[pl.ds(off, CHUNK)], srcb.at[p],
                              esem.at[p]).start()
        pltpu.make_async_copy(dst_hbm.at[pl.ds(off, CHUNK)], dstb.at[p],
                              esem.at[p]).start()
        pltpu.make_async_copy(w_hbm.at[pl.ds(off, CHUNK)], wb.at[p],
                              esem.at[p]).start()

    def wait_edges(g, p):
        off = g * CHUNK
        pltpu.make_async_copy(src_hbm.at[pl.ds(off, CHUNK)], srcb.at[p],
                              esem.at[p]).wait()
        pltpu.make_async_copy(dst_hbm.at[pl.ds(off, CHUNK)], dstb.at[p],
                              esem.at[p]).wait()
        pltpu.make_async_copy(w_hbm.at[pl.ds(off, CHUNK)], wb.at[p],
                              esem.at[p]).wait()

    def compute(p, table, acc):
        def body(r, carry):
            for b in range(2):
                base = r * 32 + b * 16
                src16 = srcb[p, pl.ds(base, 16)]
                dst16 = dstb[p, pl.ds(base, 16)]
                w16 = wb[p, pl.ds(base, 16)]
                src4 = src16 * 4
                didx4 = dst16 * 4
                for cc in range(CPP):
                    gv = plsc.load_gather(table, [src4 + cc])
                    plsc.addupdate_scatter(acc, [didx4 + cc], gv * w16)
            return carry
        lax.fori_loop(0, CHUNK // 32, body, 0)

    def sweep(table, acc):
        def zbody(i, carry):
            acc[pl.ds(i * 16, 16)] = jnp.zeros((16,), jnp.float32)
            return carry
        lax.fori_loop(0, HSLAB // 16, zbody, 0)

        fire_edges(0, 0)

        def unit(g, p, fire_next):
            wait_edges(g, p)
            if fire_next:
                fire_edges(g + 1, 1 - p)
            compute(p, table, acc)

        unit(0, 0, True)

        def two(i, carry):
            unit(2 * i - 1, 1, True)
            unit(2 * i, 0, True)
            return carry
        lax.fori_loop(1, NCHUNK // 2, two, 0)
        unit(NCHUNK - 1, 1, False)

    for q in range(2):
        goff = toff + q * HSLAB

        pltpu.sync_copy(ego0_hbm.at[pl.ds(goff, HSLAB)], bufa)

        sweep(bufa, bufb)
        pltpu.sync_copy(bufb, l1_hbm.at[pl.ds(goff, HSLAB)])
        sweep(bufb, bufa)
        pltpu.sync_copy(bufa, l2_hbm.at[pl.ds(goff, HSLAB)])
        sweep(bufa, bufb)

        for j in range(N_NODES // FR):
            pltpu.sync_copy(ego0_hbm.at[pl.ds(goff + j * FE, FE)], fl1)
            pltpu.sync_copy(l1_hbm.at[pl.ds(goff + j * FE, FE)], fl2)
            pltpu.sync_copy(l2_hbm.at[pl.ds(goff + j * FE, FE)], fl3)

            def fbody(r, carry):
                sl = pl.ds(r * 16, 16)
                v = fl1[sl] + fl2[sl] + fl3[sl] + bufb[pl.ds(j * FE + r * 16, 16)]
                fl1[sl] = v * 0.25
                return carry
            lax.fori_loop(0, FE // 16, fbody, 0)
            pltpu.sync_copy(fl1, out_hbm.at[pl.ds(goff + j * FE, FE)])


@jax.jit
def _run(ego0_t, src, dst, w):
    f32 = jnp.float32
    kfn = functools.partial(
        pl.kernel,
        out_type=[
            jax.ShapeDtypeStruct((NT * SLAB,), f32),
            jax.ShapeDtypeStruct((NT * SLAB,), f32),
            jax.ShapeDtypeStruct((NT * SLAB,), f32),
        ],
        scratch_types=[
            pltpu.VMEM((HSLAB,), f32),
            pltpu.VMEM((HSLAB,), f32),
            pltpu.VMEM((2, CHUNK), jnp.int32),
            pltpu.VMEM((2, CHUNK), jnp.int32),
            pltpu.VMEM((2, CHUNK), f32),
            pltpu.VMEM((FE,), f32),
            pltpu.VMEM((FE,), f32),
            pltpu.VMEM((FE,), f32),
            pltpu.SemaphoreType.DMA((2,)),
        ],
        mesh=plsc.VectorSubcoreMesh(core_axis_name="c", subcore_axis_name="s"),
        compiler_params=pltpu.CompilerParams(
            needs_layout_passes=False, use_tc_tiling_on_sc=False),
    )(_sc_body)
    _l1, _l2, out = kfn(ego0_t, src, dst, w)
    return (out.reshape(NT, 2, N_NODES, CPP).transpose(2, 0, 1, 3)
            .reshape(N_NODES, EMB))


def kernel(user_emb, item_emb, edge_index, edge_weight):
    ego = jnp.concatenate([user_emb, item_emb], axis=0)
    ego0_t = ego.reshape(N_NODES, NT, 2, CPP).transpose(1, 2, 0, 3).reshape(-1)
    out = _run(ego0_t, edge_index[0], edge_index[1], edge_weight)
    return (out[:USER_N], out[USER_N:])


# speedup vs baseline: 4.2323x; 4.2323x over previous
import functools
import jax
import jax.numpy as jnp
from jax import lax
from jax.experimental import pallas as pl
from jax.experimental.pallas import tpu as pltpu
from jax.experimental.pallas import tpu_sc as plsc

USER_N = 5000
ITEM_N = 5000
N_NODES = USER_N + ITEM_N
N_EDGES = 160000
EMB = 256
N_LAYERS = 3

CPT = 8
CPP = 4
NT = 32
SLAB = N_NODES * CPT
HSLAB = N_NODES * CPP
CHUNK = 1600
NCHUNK = N_EDGES // CHUNK
FR = 500
FE = FR * CPP


def _sc_body(ego0_hbm, src_hbm, dst_hbm, w_hbm,
             l1_hbm, l2_hbm, out_hbm,
             bufa, bufb, srcb, dstb, wb, fl1, fl2, fl3,
             esem):
    c = lax.axis_index("c")
    s = lax.axis_index("s")
    t = c * 16 + s
    toff = t * SLAB
    iota = lax.iota(jnp.int32, 16)
    three = jnp.full((16,), 3, jnp.int32)

    def rowcol(lf):
        return lax.shift_right_logical(lf, 2), lax.bitwise_and(lf, three)

    def fire_edges(g, p):
        off = g * CHUNK
        pltpu.make_async_copy(src_hbm.at[pl.ds(off, CHUNK)], srcb.at[p],
                              esem.at[p]).start()
        pltpu.make_async_copy(dst_hbm.at[pl.ds(off, CHUNK)], dstb.at[p],
                              esem.at[p]).start()
        pltpu.make_async_copy(w_hbm.at[pl.ds(off, CHUNK)], wb.at[p],
                              esem.at[p]).start()

    def wait_edges(g, p):
        off = g * CHUNK
        pltpu.make_async_copy(src_hbm.at[pl.ds(off, CHUNK)], srcb.at[p],
                              esem.at[p]).wait()
        pltpu.make_async_copy(dst_hbm.at[pl.ds(off, CHUNK)], dstb.at[p],
                              esem.at[p]).wait()
        pltpu.make_async_copy(w_hbm.at[pl.ds(off, CHUNK)], wb.at[p],
                              esem.at[p]).wait()

    def compute(p, table, acc):
        @plsc.parallel_loop(0, CHUNK // 16, 1, unroll=8)
        def body(r):
            base = r * 16
            src16 = srcb[p, pl.ds(base, 16)]
            dst16 = dstb[p, pl.ds(base, 16)]
            w16 = wb[p, pl.ds(base, 16)]
            for cc in range(CPP):
                gv = plsc.load_gather(table, [src16 + cc * N_NODES])
                plsc.addupdate_scatter(acc, [dst16 + cc * N_NODES], gv * w16)

    def sweep(table, acc):
        @plsc.parallel_loop(0, HSLAB // 16, 1, unroll=8)
        def zbody(i):
            acc[pl.ds(i * 16, 16)] = jnp.zeros((16,), jnp.float32)

        fire_edges(0, 0)

        def unit(g, p, fire_next):
            wait_edges(g, p)
            if fire_next:
                fire_edges(g + 1, 1 - p)
            compute(p, table, acc)

        unit(0, 0, True)

        def two(i, carry):
            unit(2 * i - 1, 1, True)
            unit(2 * i, 0, True)
            return carry
        lax.fori_loop(1, NCHUNK // 2, two, 0)
        unit(NCHUNK - 1, 1, False)

    for q in range(2):
        goff = toff + q * HSLAB

        pltpu.sync_copy(ego0_hbm.at[pl.ds(goff, HSLAB)], bufa)

        sweep(bufa, bufb)
        pltpu.sync_copy(bufb, l1_hbm.at[pl.ds(goff, HSLAB)])
        sweep(bufb, bufa)
        pltpu.sync_copy(bufa, l2_hbm.at[pl.ds(goff, HSLAB)])
        sweep(bufa, bufb)

        for j in range(N_NODES // FR):
            pltpu.sync_copy(ego0_hbm.at[pl.ds(goff + j * FE, FE)], fl1)
            pltpu.sync_copy(l1_hbm.at[pl.ds(goff + j * FE, FE)], fl2)
            pltpu.sync_copy(l2_hbm.at[pl.ds(goff + j * FE, FE)], fl3)

            def fbody(r, carry):
                sl = pl.ds(r * 16, 16)
                v = fl1[sl] + fl2[sl] + fl3[sl] + bufb[pl.ds(j * FE + r * 16, 16)]
                fl1[sl] = v * 0.25
                return carry
            lax.fori_loop(0, FE // 16, fbody, 0)
            pltpu.sync_copy(fl1, out_hbm.at[pl.ds(goff + j * FE, FE)])


@jax.jit
def _run(ego0_t, src, dst, w):
    f32 = jnp.float32
    kfn = functools.partial(
        pl.kernel,
        out_type=[
            jax.ShapeDtypeStruct((NT * SLAB,), f32),
            jax.ShapeDtypeStruct((NT * SLAB,), f32),
            jax.ShapeDtypeStruct((NT * SLAB,), f32),
        ],
        scratch_types=[
            pltpu.VMEM((HSLAB,), f32),
            pltpu.VMEM((HSLAB,), f32),
            pltpu.VMEM((2, CHUNK), jnp.int32),
            pltpu.VMEM((2, CHUNK), jnp.int32),
            pltpu.VMEM((2, CHUNK), f32),
            pltpu.VMEM((FE,), f32),
            pltpu.VMEM((FE,), f32),
            pltpu.VMEM((FE,), f32),
            pltpu.SemaphoreType.DMA((2,)),
        ],
        mesh=plsc.VectorSubcoreMesh(core_axis_name="c", subcore_axis_name="s"),
        compiler_params=pltpu.CompilerParams(
            needs_layout_passes=False, use_tc_tiling_on_sc=False),
    )(_sc_body)
    _l1, _l2, out = kfn(ego0_t, src, dst, w)
    return (out.reshape(NT, 2, CPP, N_NODES).transpose(3, 0, 1, 2)
            .reshape(N_NODES, EMB))


def kernel(user_emb, item_emb, edge_index, edge_weight):
    ego = jnp.concatenate([user_emb, item_emb], axis=0)
    ego0_t = ego.reshape(N_NODES, NT, 2, CPP).transpose(1, 2, 3, 0).reshape(-1)
    out = _run(ego0_t, edge_index[0], edge_index[1], edge_weight)
    return (out[:USER_N], out[USER_N:])


# speedup vs baseline: 4.4371x; 1.0484x over previous
import functools
import jax
import jax.numpy as jnp
from jax import lax
from jax.experimental import pallas as pl
from jax.experimental.pallas import tpu as pltpu
from jax.experimental.pallas import tpu_sc as plsc

USER_N = 5000
ITEM_N = 5000
N_NODES = USER_N + ITEM_N
N_EDGES = 160000
EMB = 256
N_LAYERS = 3

CPT = 8
CPP = 4
NT = 32
SLAB = N_NODES * CPT
HSLAB = N_NODES * CPP
CHUNK = 3200
NCHUNK = N_EDGES // CHUNK
FR = 500
FE = FR * CPP


def _sc_body(ego0_hbm, src_hbm, dst_hbm, w_hbm,
             l1_hbm, l2_hbm, out_hbm,
             bufa, bufb, srcb, dstb, wb, fl1, fl2, fl3,
             esem):
    c = lax.axis_index("c")
    s = lax.axis_index("s")
    t = c * 16 + s
    toff = t * SLAB
    iota = lax.iota(jnp.int32, 16)
    three = jnp.full((16,), 3, jnp.int32)

    def rowcol(lf):
        return lax.shift_right_logical(lf, 2), lax.bitwise_and(lf, three)

    def fire_edges(g, p):
        off = g * CHUNK
        pltpu.make_async_copy(src_hbm.at[pl.ds(off, CHUNK)], srcb.at[p],
                              esem.at[p]).start()
        pltpu.make_async_copy(dst_hbm.at[pl.ds(off, CHUNK)], dstb.at[p],
                              esem.at[p]).start()
        pltpu.make_async_copy(w_hbm.at[pl.ds(off, CHUNK)], wb.at[p],
                              esem.at[p]).start()

    def wait_edges(g, p):
        off = g * CHUNK
        pltpu.make_async_copy(src_hbm.at[pl.ds(off, CHUNK)], srcb.at[p],
                              esem.at[p]).wait()
        pltpu.make_async_copy(dst_hbm.at[pl.ds(off, CHUNK)], dstb.at[p],
                              esem.at[p]).wait()
        pltpu.make_async_copy(w_hbm.at[pl.ds(off, CHUNK)], wb.at[p],
                              esem.at[p]).wait()

    def compute(p, table, acc):
        @plsc.parallel_loop(0, CHUNK // 16, 1, unroll=10)
        def body(r):
            base = r * 16
            src16 = srcb[p, pl.ds(base, 16)]
            dst16 = dstb[p, pl.ds(base, 16)]
            w16 = wb[p, pl.ds(base, 16)]
            for cc in range(CPP):
                gv = plsc.load_gather(table, [src16 + cc * N_NODES])
                plsc.addupdate_scatter(acc, [dst16 + cc * N_NODES], gv * w16)

    def sweep(table, acc):
        @plsc.parallel_loop(0, HSLAB // 16, 1, unroll=8)
        def zbody(i):
            acc[pl.ds(i * 16, 16)] = jnp.zeros((16,), jnp.float32)

        fire_edges(0, 0)

        def unit(g, p, fire_next):
            wait_edges(g, p)
            if fire_next:
                fire_edges(g + 1, 1 - p)
            compute(p, table, acc)

        unit(0, 0, True)

        def two(i, carry):
            unit(2 * i - 1, 1, True)
            unit(2 * i, 0, True)
            return carry
        lax.fori_loop(1, NCHUNK // 2, two, 0)
        unit(NCHUNK - 1, 1, False)

    for q in range(2):
        goff = toff + q * HSLAB

        pltpu.sync_copy(ego0_hbm.at[pl.ds(goff, HSLAB)], bufa)

        sweep(bufa, bufb)
        pltpu.sync_copy(bufb, l1_hbm.at[pl.ds(goff, HSLAB)])
        sweep(bufb, bufa)
        pltpu.sync_copy(bufa, l2_hbm.at[pl.ds(goff, HSLAB)])
        sweep(bufa, bufb)

        for j in range(N_NODES // FR):
            pltpu.sync_copy(ego0_hbm.at[pl.ds(goff + j * FE, FE)], fl1)
            pltpu.sync_copy(l1_hbm.at[pl.ds(goff + j * FE, FE)], fl2)
            pltpu.sync_copy(l2_hbm.at[pl.ds(goff + j * FE, FE)], fl3)

            def fbody(r, carry):
                sl = pl.ds(r * 16, 16)
                v = fl1[sl] + fl2[sl] + fl3[sl] + bufb[pl.ds(j * FE + r * 16, 16)]
                fl1[sl] = v * 0.25
                return carry
            lax.fori_loop(0, FE // 16, fbody, 0)
            pltpu.sync_copy(fl1, out_hbm.at[pl.ds(goff + j * FE, FE)])


@jax.jit
def _run(ego0_t, src, dst, w):
    f32 = jnp.float32
    kfn = functools.partial(
        pl.kernel,
        out_type=[
            jax.ShapeDtypeStruct((NT * SLAB,), f32),
            jax.ShapeDtypeStruct((NT * SLAB,), f32),
            jax.ShapeDtypeStruct((NT * SLAB,), f32),
        ],
        scratch_types=[
            pltpu.VMEM((HSLAB,), f32),
            pltpu.VMEM((HSLAB,), f32),
            pltpu.VMEM((2, CHUNK), jnp.int32),
            pltpu.VMEM((2, CHUNK), jnp.int32),
            pltpu.VMEM((2, CHUNK), f32),
            pltpu.VMEM((FE,), f32),
            pltpu.VMEM((FE,), f32),
            pltpu.VMEM((FE,), f32),
            pltpu.SemaphoreType.DMA((2,)),
        ],
        mesh=plsc.VectorSubcoreMesh(core_axis_name="c", subcore_axis_name="s"),
        compiler_params=pltpu.CompilerParams(
            needs_layout_passes=False, use_tc_tiling_on_sc=False),
    )(_sc_body)
    _l1, _l2, out = kfn(ego0_t, src, dst, w)
    return (out.reshape(NT, 2, CPP, N_NODES).transpose(3, 0, 1, 2)
            .reshape(N_NODES, EMB))


def kernel(user_emb, item_emb, edge_index, edge_weight):
    ego = jnp.concatenate([user_emb, item_emb], axis=0)
    ego0_t = ego.reshape(N_NODES, NT, 2, CPP).transpose(1, 2, 3, 0).reshape(-1)
    out = _run(ego0_t, edge_index[0], edge_index[1], edge_weight)
    return (out[:USER_N], out[USER_N:])


# speedup vs baseline: 4.9199x; 1.1088x over previous
import functools
import jax
import jax.numpy as jnp
from jax import lax
from jax.experimental import pallas as pl
from jax.experimental.pallas import tpu as pltpu
from jax.experimental.pallas import tpu_sc as plsc

USER_N = 5000
ITEM_N = 5000
N_NODES = USER_N + ITEM_N
N_EDGES = 160000
EMB = 256
N_LAYERS = 3

CPT = 8
CPP = 4
NT = 32
SLAB = N_NODES * CPT
HSLAB = N_NODES * CPP
CHUNK = 3200
NCHUNK = N_EDGES // CHUNK
FR = 500
FE = FR * CPP


def _sc_body(ego0_hbm, src_hbm, dst_hbm, w_hbm,
             l1_hbm, l2_hbm, out_hbm,
             bufa, bufp, srcb, dstb, wb, fl1, fl2, fl3,
             esem):
    c = lax.axis_index("c")
    s = lax.axis_index("s")
    t = c * 16 + s
    toff = t * SLAB
    iota = lax.iota(jnp.int32, 16)
    three = jnp.full((16,), 3, jnp.int32)

    def rowcol(lf):
        return lax.shift_right_logical(lf, 2), lax.bitwise_and(lf, three)

    def fire_edges(g, p):
        off = g * CHUNK
        pltpu.make_async_copy(src_hbm.at[pl.ds(off, CHUNK)], srcb.at[p],
                              esem.at[p]).start()
        pltpu.make_async_copy(dst_hbm.at[pl.ds(off, CHUNK)], dstb.at[p],
                              esem.at[p]).start()
        pltpu.make_async_copy(w_hbm.at[pl.ds(off, CHUNK)], wb.at[p],
                              esem.at[p]).start()

    def wait_edges(g, p):
        off = g * CHUNK
        pltpu.make_async_copy(src_hbm.at[pl.ds(off, CHUNK)], srcb.at[p],
                              esem.at[p]).wait()
        pltpu.make_async_copy(dst_hbm.at[pl.ds(off, CHUNK)], dstb.at[p],
                              esem.at[p]).wait()
        pltpu.make_async_copy(w_hbm.at[pl.ds(off, CHUNK)], wb.at[p],
                              esem.at[p]).wait()

    def compute(p, tableP, acc):
        @plsc.parallel_loop(0, CHUNK // 16, 1, unroll=10)
        def body(r):
            base = r * 16
            src16 = srcb[p, pl.ds(base, 16)]
            dst16 = dstb[p, pl.ds(base, 16)]
            w16 = wb[p, pl.ds(base, 16)]
            for h in range(CPP // 2):
                pw = plsc.load_gather(tableP, [src16 + h * N_NODES])
                av, bv = plsc.unpack(plsc.bitcast(pw, jnp.bfloat16),
                                     format=plsc.PackFormat.INTERLEAVED)
                plsc.addupdate_scatter(acc, [dst16 + 2 * h * N_NODES],
                                       av * w16)
                plsc.addupdate_scatter(acc, [dst16 + (2 * h + 1) * N_NODES],
                                       bv * w16)

    def sweep(table, acc):
        @plsc.parallel_loop(0, HSLAB // 16, 1, unroll=8)
        def zbody(i):
            acc[pl.ds(i * 16, 16)] = jnp.zeros((16,), jnp.float32)

        fire_edges(0, 0)

        def unit(g, p, fire_next):
            wait_edges(g, p)
            if fire_next:
                fire_edges(g + 1, 1 - p)
            compute(p, table, acc)

        unit(0, 0, True)

        def two(i, carry):
            unit(2 * i - 1, 1, True)
            unit(2 * i, 0, True)
            return carry
        lax.fori_loop(1, NCHUNK // 2, two, 0)
        unit(NCHUNK - 1, 1, False)

    def repack(acc, tableP):
        @plsc.parallel_loop(0, N_NODES // 16, 1, unroll=4)
        def rbody(i):
            i16 = i * 16
            for h in range(CPP // 2):
                lo = acc[pl.ds(2 * h * N_NODES + i16, 16)]
                hi = acc[pl.ds((2 * h + 1) * N_NODES + i16, 16)]
                pk = plsc.pack(lo, hi, format=plsc.PackFormat.INTERLEAVED)
                tableP[pl.ds(h * N_NODES + i16, 16)] = plsc.bitcast(
                    pk, jnp.int32)

    for q in range(2):
        goff = toff + q * HSLAB

        pltpu.sync_copy(ego0_hbm.at[pl.ds(goff, HSLAB)], bufa)
        repack(bufa, bufp)

        sweep(bufp, bufa)
        pltpu.sync_copy(bufa, l1_hbm.at[pl.ds(goff, HSLAB)])
        repack(bufa, bufp)
        sweep(bufp, bufa)
        pltpu.sync_copy(bufa, l2_hbm.at[pl.ds(goff, HSLAB)])
        repack(bufa, bufp)
        sweep(bufp, bufa)

        for j in range(N_NODES // FR):
            pltpu.sync_copy(ego0_hbm.at[pl.ds(goff + j * FE, FE)], fl1)
            pltpu.sync_copy(l1_hbm.at[pl.ds(goff + j * FE, FE)], fl2)
            pltpu.sync_copy(l2_hbm.at[pl.ds(goff + j * FE, FE)], fl3)

            def fbody(r, carry):
                sl = pl.ds(r * 16, 16)
                v = fl1[sl] + fl2[sl] + fl3[sl] + bufa[pl.ds(j * FE + r * 16, 16)]
                fl1[sl] = v * 0.25
                return carry
            lax.fori_loop(0, FE // 16, fbody, 0)
            pltpu.sync_copy(fl1, out_hbm.at[pl.ds(goff + j * FE, FE)])


@jax.jit
def _run(ego0_t, src, dst, w):
    f32 = jnp.float32
    kfn = functools.partial(
        pl.kernel,
        out_type=[
            jax.ShapeDtypeStruct((NT * SLAB,), f32),
            jax.ShapeDtypeStruct((NT * SLAB,), f32),
            jax.ShapeDtypeStruct((NT * SLAB,), f32),
        ],
        scratch_types=[
            pltpu.VMEM((HSLAB,), f32),
            pltpu.VMEM((HSLAB // 2,), jnp.int32),
            pltpu.VMEM((2, CHUNK), jnp.int32),
            pltpu.VMEM((2, CHUNK), jnp.int32),
            pltpu.VMEM((2, CHUNK), f32),
            pltpu.VMEM((FE,), f32),
            pltpu.VMEM((FE,), f32),
            pltpu.VMEM((FE,), f32),
            pltpu.SemaphoreType.DMA((2,)),
        ],
        mesh=plsc.VectorSubcoreMesh(core_axis_name="c", subcore_axis_name="s"),
        compiler_params=pltpu.CompilerParams(
            needs_layout_passes=False, use_tc_tiling_on_sc=False),
    )(_sc_body)
    _l1, _l2, out = kfn(ego0_t, src, dst, w)
    return (out.reshape(NT, 2, CPP, N_NODES).transpose(3, 0, 1, 2)
            .reshape(N_NODES, EMB))


def kernel(user_emb, item_emb, edge_index, edge_weight):
    ego = jnp.concatenate([user_emb, item_emb], axis=0)
    ego0_t = ego.reshape(N_NODES, NT, 2, CPP).transpose(1, 2, 3, 0).reshape(-1)
    out = _run(ego0_t, edge_index[0], edge_index[1], edge_weight)
    return (out[:USER_N], out[USER_N:])


# speedup vs baseline: 4.9206x; 1.0001x over previous
import functools
import jax
import jax.numpy as jnp
from jax import lax
from jax.experimental import pallas as pl
from jax.experimental.pallas import tpu as pltpu
from jax.experimental.pallas import tpu_sc as plsc

USER_N = 5000
ITEM_N = 5000
N_NODES = USER_N + ITEM_N
N_EDGES = 160000
EMB = 256
N_LAYERS = 3

CPT = 8
CPP = 4
NT = 32
SLAB = N_NODES * CPT
HSLAB = N_NODES * CPP
CHUNK = 3200
NCHUNK = N_EDGES // CHUNK
FR = 500
FE = FR * CPP


def _sc_body(ego0_hbm, src_hbm, dst_hbm, w_hbm,
             l1_hbm, l2_hbm, out_hbm,
             bufa, bufp, srcb, dstb, wb, fl1, fl2, fl3,
             esem):
    c = lax.axis_index("c")
    s = lax.axis_index("s")
    t = c * 16 + s
    toff = t * SLAB

    def fire_edges(g, p):
        off = g * CHUNK
        pltpu.make_async_copy(src_hbm.at[pl.ds(off, CHUNK)], srcb.at[p],
                              esem.at[p]).start()
        pltpu.make_async_copy(dst_hbm.at[pl.ds(off, CHUNK)], dstb.at[p],
                              esem.at[p]).start()
        pltpu.make_async_copy(w_hbm.at[pl.ds(off, CHUNK)], wb.at[p],
                              esem.at[p]).start()

    def wait_edges(g, p):
        off = g * CHUNK
        pltpu.make_async_copy(src_hbm.at[pl.ds(off, CHUNK)], srcb.at[p],
                              esem.at[p]).wait()
        pltpu.make_async_copy(dst_hbm.at[pl.ds(off, CHUNK)], dstb.at[p],
                              esem.at[p]).wait()
        pltpu.make_async_copy(w_hbm.at[pl.ds(off, CHUNK)], wb.at[p],
                              esem.at[p]).wait()

    def compute(p, tableP, acc):
        @plsc.parallel_loop(0, CHUNK // 16, 1, unroll=10)
        def body(r):
            base = r * 16
            src16 = srcb[p, pl.ds(base, 16)]
            dst16 = dstb[p, pl.ds(base, 16)]
            w16 = wb[p, pl.ds(base, 16)]
            for h in range(CPP // 2):
                pw = plsc.load_gather(tableP, [src16 + h * N_NODES])
                av, bv = plsc.unpack(plsc.bitcast(pw, jnp.bfloat16),
                                     format=plsc.PackFormat.INTERLEAVED)
                plsc.addupdate_scatter(acc, [dst16 + 2 * h * N_NODES],
                                       av * w16)
                plsc.addupdate_scatter(acc, [dst16 + (2 * h + 1) * N_NODES],
                                       bv * w16)

    def sweep(table, acc):
        @plsc.parallel_loop(0, HSLAB // 16, 1, unroll=8)
        def zbody(i):
            acc[pl.ds(i * 16, 16)] = jnp.zeros((16,), jnp.float32)

        fire_edges(0, 0)

        def unit(g, p, fire_next):
            wait_edges(g, p)
            if fire_next:
                fire_edges(g + 1, 1 - p)
            compute(p, table, acc)

        unit(0, 0, True)

        def two(i, carry):
            unit(2 * i - 1, 1, True)
            unit(2 * i, 0, True)
            return carry
        lax.fori_loop(1, NCHUNK // 2, two, 0)
        unit(NCHUNK - 1, 1, False)

    def repack(acc, tableP):
        @plsc.parallel_loop(0, N_NODES // 16, 1, unroll=4)
        def rbody(i):
            i16 = i * 16
            for h in range(CPP // 2):
                lo = acc[pl.ds(2 * h * N_NODES + i16, 16)]
                hi = acc[pl.ds((2 * h + 1) * N_NODES + i16, 16)]
                pk = plsc.pack(lo, hi, format=plsc.PackFormat.INTERLEAVED)
                tableP[pl.ds(h * N_NODES + i16, 16)] = plsc.bitcast(
                    pk, jnp.int32)

    for q in range(2):
        goff = toff + q * HSLAB

        pltpu.sync_copy(ego0_hbm.at[pl.ds(goff, HSLAB)], bufa)
        repack(bufa, bufp)

        sweep(bufp, bufa)
        pltpu.sync_copy(bufa, l1_hbm.at[pl.ds(goff, HSLAB)])
        repack(bufa, bufp)
        sweep(bufp, bufa)
        pltpu.sync_copy(bufa, l2_hbm.at[pl.ds(goff, HSLAB)])
        repack(bufa, bufp)
        sweep(bufp, bufa)

        for j in range(N_NODES // FR):
            pltpu.sync_copy(ego0_hbm.at[pl.ds(goff + j * FE, FE)], fl1)
            pltpu.sync_copy(l1_hbm.at[pl.ds(goff + j * FE, FE)], fl2)
            pltpu.sync_copy(l2_hbm.at[pl.ds(goff + j * FE, FE)], fl3)

            def fbody(r, carry):
                sl = pl.ds(r * 16, 16)
                v = fl1[sl] + fl2[sl] + fl3[sl] + bufa[pl.ds(j * FE + r * 16, 16)]
                fl1[sl] = v * 0.25
                return carry
            lax.fori_loop(0, FE // 16, fbody, 0)
            pltpu.sync_copy(fl1, out_hbm.at[pl.ds(goff + j * FE, FE)])


@jax.jit
def _run(ego0_t, src, dst, w):
    f32 = jnp.float32
    kfn = functools.partial(
        pl.kernel,
        out_type=[
            jax.ShapeDtypeStruct((NT * SLAB,), f32),
            jax.ShapeDtypeStruct((NT * SLAB,), f32),
            jax.ShapeDtypeStruct((NT * SLAB,), f32),
        ],
        scratch_types=[
            pltpu.VMEM((HSLAB,), f32),
            pltpu.VMEM((HSLAB // 2,), jnp.int32),
            pltpu.VMEM((2, CHUNK), jnp.int32),
            pltpu.VMEM((2, CHUNK), jnp.int32),
            pltpu.VMEM((2, CHUNK), f32),
            pltpu.VMEM((FE,), f32),
            pltpu.VMEM((FE,), f32),
            pltpu.VMEM((FE,), f32),
            pltpu.SemaphoreType.DMA((2,)),
        ],
        mesh=plsc.VectorSubcoreMesh(core_axis_name="c", subcore_axis_name="s"),
        compiler_params=pltpu.CompilerParams(
            needs_layout_passes=False, use_tc_tiling_on_sc=False),
    )(_sc_body)
    _l1, _l2, out = kfn(ego0_t, src, dst, w)
    return (out.reshape(NT, 2, CPP, N_NODES).transpose(3, 0, 1, 2)
            .reshape(N_NODES, EMB))


def kernel(user_emb, item_emb, edge_index, edge_weight):
    ego = jnp.concatenate([user_emb, item_emb], axis=0)
    ego0_t = ego.reshape(N_NODES, NT, 2, CPP).transpose(1, 2, 3, 0).reshape(-1)
    out = _run(ego0_t, edge_index[0], edge_index[1], edge_weight)
    return (out[:USER_N], out[USER_N:])
